# Initial kernel scaffold; baseline (speedup 1.0000x reference)
import jax
import jax.numpy as jnp
from jax.experimental import pallas as pl


def kernel(x, edge_index, edge_weight, batch, W1, b1, Wp_rel, bp, Wp_root, W2, b2, W_ih, W_hh, b_ih, b_hh, Wo, bo):
    raise NotImplementedError("write your pallas kernel here")


# speedup vs baseline: 29.0475x; 29.0475x over previous
import functools

import jax
import jax.numpy as jnp
from jax import lax
from jax.experimental import pallas as pl
from jax.experimental.pallas import tpu as pltpu
from jax.experimental.pallas import tpu_sc as plsc

N = 10000
E = 320000
T = 8
B = 64
PC = 16
H = 64
RATIO = 0.8

_NC = 2
_NS = 16
_NPAD = 10240
_EPW = E // (_NC * _NS)
_K = 80
_ITERS = _EPW // _K
_RPT = _NPAD // _NS


@functools.lru_cache(maxsize=None)
def _edge_pass_fn(D, scale, gather):
    nvr = D // 16
    mesh = plsc.VectorSubcoreMesh(core_axis_name="c", subcore_axis_name="s")

    scratch = [pltpu.VMEM((_K,), jnp.int32)]
    if gather:
        scratch.append(pltpu.VMEM((_K,), jnp.int32))
    if scale:
        scratch.append(pltpu.VMEM((_K,), jnp.float32))
    scratch += [
        pltpu.VMEM((_K, D), jnp.float32),
        pltpu.VMEM_SHARED((_NPAD, D), jnp.float32),
        pltpu.SemaphoreType.DMA,
    ]

    def body(*refs):
        refs = list(refs)
        table = refs.pop(0) if gather else None
        srcr = refs.pop(0) if gather else None
        dstr = refs.pop(0)
        ewr = refs.pop(0) if scale else None
        out = refs.pop(0)
        idx_d = refs.pop(0)
        idx_s = refs.pop(0) if gather else None
        ew_v = refs.pop(0) if scale else None
        rows = refs.pop(0)
        acc = refs.pop(0)
        sem = refs.pop(0)

        c = lax.axis_index("c")
        s = lax.axis_index("s")
        ebase = (c * _NS + s) * _EPW
        tbase = s * _RPT

        def zrow(e, carry):
            for j in range(nvr):
                rows[e, pl.ds(j * 16, 16)] = jnp.zeros((16,), jnp.float32)
            return carry
        lax.fori_loop(0, _K, zrow, 0)
        for kc in range(_RPT // _K):
            pltpu.sync_copy(rows, acc.at[pl.ds(tbase + kc * _K, _K)])
        plsc.subcore_barrier()

        zi16 = jnp.zeros((16,), jnp.int32)
        lane0 = lax.iota(jnp.int32, 16) == 0

        def step(it, carry):
            base = ebase + it * _K
            pltpu.sync_copy(dstr.at[pl.ds(base, _K)], idx_d)
            if gather:
                pltpu.sync_copy(srcr.at[pl.ds(base, _K)], idx_s)
            if scale:
                pltpu.sync_copy(ewr.at[pl.ds(base, _K)], ew_v)
            if gather:
                pltpu.async_copy(table.at[idx_s], rows, sem).wait()
            if gather and scale:
                def sc_body(g, cr):
                    wvec = ew_v[pl.ds(g * 16, 16)]
                    for l in range(16):
                        e = g * 16 + l
                        w = wvec[l]
                        for j in range(nvr):
                            sl = pl.ds(j * 16, 16)
                            rows[e, sl] = rows[e, sl] * w
                    return cr
                lax.fori_loop(0, _K // 16, sc_body, 0)
            if not gather:
                def a_body(g, cr):
                    wvec = ew_v[pl.ds(g * 16, 16)]
                    for l in range(16):
                        e = g * 16 + l
                        rows[e, pl.ds(0, 16)] = jnp.where(lane0, wvec[l], 0.0)
                    return cr
                lax.fori_loop(0, _K // 16, a_body, 0)
            pltpu.sync_copy(rows, acc.at[idx_d], add=True)
            return carry
        lax.fori_loop(0, _ITERS, step, 0)
        plsc.subcore_barrier()
        pltpu.sync_copy(acc.at[pl.ds(tbase, _RPT)],
                        out.at[c, pl.ds(tbase, _RPT)])

    return pl.kernel(
        body, mesh=mesh,
        out_type=jax.ShapeDtypeStruct((_NC, _NPAD, D), jnp.float32),
        scratch_types=scratch,
        compiler_params=pltpu.CompilerParams(use_tc_tiling_on_sc=False),
    )


def _edge_pass(table, src, dst, ew, D, scale, gather):
    fn = _edge_pass_fn(D, scale, gather)
    args = []
    if gather:
        args += [table, src]
    args += [dst]
    if scale:
        args += [ew]
    out = fn(*args)
    return out[0] + out[1]


def _pad_rows(a):
    return jnp.pad(a, ((0, _NPAD - N), (0, 0)))


def _lstm_body(seq_ref, wih_ref, whh_ref, bias_ref, wo_ref, bo_ref, out_ref):
    hs = jnp.zeros((B, H), jnp.float32)
    cs = jnp.zeros((B, H), jnp.float32)
    wih = wih_ref[...]
    whh = whh_ref[...]
    bias = bias_ref[...]
    for t in range(T):
        g = (jnp.dot(seq_ref[:, t, :], wih, preferred_element_type=jnp.float32)
             + jnp.dot(hs, whh, preferred_element_type=jnp.float32) + bias)
        i = jax.nn.sigmoid(g[:, 0:H])
        f = jax.nn.sigmoid(g[:, H:2 * H])
        gg = jnp.tanh(g[:, 2 * H:3 * H])
        o = jax.nn.sigmoid(g[:, 3 * H:4 * H])
        cs = f * cs + i * gg
        hs = o * jnp.tanh(cs)
    out_ref[...] = jax.nn.sigmoid(
        jnp.dot(hs, wo_ref[...], preferred_element_type=jnp.float32) + bo_ref[...])


def _lstm_head(seq, W_ih, W_hh, b_ih, b_hh, Wo, bo):
    bias = (b_ih + b_hh).reshape(1, 4 * H)
    return pl.pallas_call(
        _lstm_body,
        out_shape=jax.ShapeDtypeStruct((B, 1), jnp.float32),
    )(seq, W_ih.T, W_hh.T, bias, Wo, bo.reshape(1, 1))


_HI = jax.lax.Precision.HIGHEST


def kernel(x, edge_index, edge_weight, batch, W1, b1, Wp_rel, bp, Wp_root,
           W2, b2, W_ih, W_hh, b_ih, b_hh, Wo, bo):
    src = edge_index[0]
    dst = edge_index[1]
    ew = edge_weight
    counts = jnp.bincount(batch, length=B)
    starts = jnp.concatenate(
        [jnp.zeros((1,), counts.dtype), jnp.cumsum(counts)])[:-1]
    k_per = jnp.ceil(RATIO * counts.astype(jnp.float32)).astype(jnp.int32)

    degA = _edge_pass(None, src, dst, ew, 16, True, False)
    deg1 = degA[:N, 0] + 1.0
    dinv1 = lax.rsqrt(deg1)

    XW = jnp.einsum('tnf,fp->ntp', x, W1, precision=_HI)
    M1 = dinv1[:, None, None] * XW
    Z1 = _edge_pass(_pad_rows(M1.reshape(N, T * PC)), src, dst, ew,
                    128, True, True)[:N].reshape(N, T, PC)
    h1 = jax.nn.relu(dinv1[:, None, None] * Z1
                     + (dinv1 ** 2)[:, None, None] * XW + b1)

    r = jnp.einsum('ntp,p->nt', h1, Wp_rel[:, 0], precision=_HI)
    rt = jnp.pad(r, ((0, _NPAD - N), (0, 16 - T)))
    aggr = _edge_pass(rt, src, dst, ew, 16, False, True)[:N, :T]
    score = jnp.tanh(aggr + bp[0]
                     + jnp.einsum('ntp,p->nt', h1, Wp_root[:, 0], precision=_HI))

    masks = []
    arange_n = jnp.arange(N)
    for t in range(T):
        order = jnp.lexsort((-score[:, t], batch))
        b_sorted = batch[order]
        rank = arange_n - starts[b_sorted]
        keep = (rank < k_per[b_sorted]).astype(jnp.float32)
        masks.append(jnp.zeros((N,), jnp.float32).at[order].set(keep))
    mask = jnp.stack(masks, 1)

    xp = h1 * score[:, :, None] * mask[:, :, None]

    maskt = jnp.pad(mask, ((0, _NPAD - N), (0, 16 - T)))
    DEG2 = _edge_pass(maskt, src, dst, ew, 16, True, True)[:N, :T]
    deg2 = mask * DEG2 + mask
    dinv2 = mask * lax.rsqrt(jnp.maximum(deg2, 1.0))

    G = dinv2[:, :, None] * xp
    Z2 = _edge_pass(_pad_rows(G.reshape(N, T * PC)), src, dst, ew,
                    128, True, True)[:N].reshape(N, T, PC)
    pre = dinv2[:, :, None] * Z2 + (dinv2 ** 2 * mask)[:, :, None] * xp
    h2 = jax.nn.relu(
        (jnp.einsum('ntp,ph->nth', pre, W2, precision=_HI) + b2)
        * mask[:, :, None])

    sums = jax.ops.segment_sum(h2, batch, num_segments=B)
    cnt = jax.ops.segment_sum(mask, batch, num_segments=B)
    seq = sums / jnp.maximum(cnt, 1.0)[:, :, None]

    return _lstm_head(seq, W_ih, W_hh, b_ih, b_hh, Wo, bo)


# speedup vs baseline: 54.4641x; 1.8750x over previous
import functools

import jax
import jax.numpy as jnp
from jax import lax
from jax.experimental import pallas as pl
from jax.experimental.pallas import tpu as pltpu
from jax.experimental.pallas import tpu_sc as plsc

N = 10000
E = 320000
T = 8
B = 64
F_IN = 128
PC = 16
H = 64
RATIO = 0.8

_NC = 2
_NS = 16
_NPAD = 10240
_EPW = E // (_NC * _NS)
_K = 80
_ITERS = _EPW // _K
_RPT = _NPAD // _NS


@functools.lru_cache(maxsize=None)
def _edge_pass_fn(D, scale, gather):
    nvr = D // 16
    mesh = plsc.VectorSubcoreMesh(core_axis_name="c", subcore_axis_name="s")

    scratch = [pltpu.VMEM((_K,), jnp.int32)]
    if gather:
        scratch.append(pltpu.VMEM((_K,), jnp.int32))
    if scale:
        scratch.append(pltpu.VMEM((_K,), jnp.float32))
    scratch += [
        pltpu.VMEM((_K, D), jnp.float32),
        pltpu.VMEM_SHARED((_NPAD, D), jnp.float32),
        pltpu.SemaphoreType.DMA,
    ]

    def body(*refs):
        refs = list(refs)
        table = refs.pop(0) if gather else None
        srcr = refs.pop(0) if gather else None
        dstr = refs.pop(0)
        ewr = refs.pop(0) if scale else None
        out = refs.pop(0)
        idx_d = refs.pop(0)
        idx_s = refs.pop(0) if gather else None
        ew_v = refs.pop(0) if scale else None
        rows = refs.pop(0)
        acc = refs.pop(0)
        sem = refs.pop(0)

        c = lax.axis_index("c")
        s = lax.axis_index("s")
        ebase = (c * _NS + s) * _EPW
        tbase = s * _RPT

        def zrow(e, carry):
            for j in range(nvr):
                rows[e, pl.ds(j * 16, 16)] = jnp.zeros((16,), jnp.float32)
            return carry
        lax.fori_loop(0, _K, zrow, 0)
        for kc in range(_RPT // _K):
            pltpu.sync_copy(rows, acc.at[pl.ds(tbase + kc * _K, _K)])
        plsc.subcore_barrier()

        zi16 = jnp.zeros((16,), jnp.int32)
        lane0 = lax.iota(jnp.int32, 16) == 0

        def step(it, carry):
            base = ebase + it * _K
            pltpu.sync_copy(dstr.at[pl.ds(base, _K)], idx_d)
            if gather:
                pltpu.sync_copy(srcr.at[pl.ds(base, _K)], idx_s)
            if scale:
                pltpu.sync_copy(ewr.at[pl.ds(base, _K)], ew_v)
            if gather:
                pltpu.async_copy(table.at[idx_s], rows, sem).wait()
            if gather and scale:
                def sc_body(g, cr):
                    wvec = ew_v[pl.ds(g * 16, 16)]
                    for l in range(16):
                        e = g * 16 + l
                        w = wvec[l]
                        for j in range(nvr):
                            sl = pl.ds(j * 16, 16)
                            rows[e, sl] = rows[e, sl] * w
                    return cr
                lax.fori_loop(0, _K // 16, sc_body, 0)
            if not gather:
                def a_body(g, cr):
                    wvec = ew_v[pl.ds(g * 16, 16)]
                    for l in range(16):
                        e = g * 16 + l
                        rows[e, pl.ds(0, 16)] = jnp.where(lane0, wvec[l], 0.0)
                    return cr
                lax.fori_loop(0, _K // 16, a_body, 0)
            pltpu.sync_copy(rows, acc.at[idx_d], add=True)
            return carry
        lax.fori_loop(0, _ITERS, step, 0)
        plsc.subcore_barrier()
        pltpu.sync_copy(acc.at[pl.ds(tbase, _RPT)],
                        out.at[c, pl.ds(tbase, _RPT)])

    return pl.kernel(
        body, mesh=mesh,
        out_type=jax.ShapeDtypeStruct((_NC, _NPAD, D), jnp.float32),
        scratch_types=scratch,
        compiler_params=pltpu.CompilerParams(use_tc_tiling_on_sc=False),
    )


def _edge_pass(table, src, dst, ew, D, scale, gather):
    fn = _edge_pass_fn(D, scale, gather)
    args = []
    if gather:
        args += [table, src]
    args += [dst]
    if scale:
        args += [ew]
    return fn(*args)


_RB2 = 400


def _row_specs(shapes):
    specs = []
    for s in shapes:
        if s[0] is None:
            full = tuple(1 if d is None else d for d in s)
            specs.append(pl.BlockSpec(
                tuple(d for d in full),
                lambda i: tuple(0 for _ in full)))
        else:
            specs.append(pl.BlockSpec(s, lambda i: (i,) + (0,) * (len(s) - 1)))
    return specs


def _prep1_body(x_ref, w1_ref, dinv_ref, xw_ref, m1_ref):
    w1 = w1_ref[...]
    dv = dinv_ref[...]
    for t in range(T):
        xw = jnp.dot(x_ref[t], w1, preferred_element_type=jnp.float32)
        xw_ref[:, t * PC:(t + 1) * PC] = xw
        m1_ref[:, t * PC:(t + 1) * PC] = dv * xw


def _prep1(x, W1, dinv1):
    return pl.pallas_call(
        _prep1_body,
        grid=(N // _RB2,),
        in_specs=[
            pl.BlockSpec((T, _RB2, F_IN), lambda i: (0, i, 0)),
            pl.BlockSpec((F_IN, PC), lambda i: (0, 0)),
            pl.BlockSpec((_RB2, 1), lambda i: (i, 0)),
        ],
        out_specs=[pl.BlockSpec((_RB2, T * PC), lambda i: (i, 0))] * 2,
        out_shape=[jax.ShapeDtypeStruct((N, T * PC), jnp.float32)] * 2,
    )(x, W1, dinv1[:, None])


def _h1r_body(z0_ref, z1_ref, xw_ref, dinv_ref, b1t_ref, wpc_ref,
              h1_ref, rs_ref):
    dv = dinv_ref[...]
    h1 = jax.nn.relu(dv * (z0_ref[...] + z1_ref[...])
                     + (dv * dv) * xw_ref[...] + b1t_ref[...])
    h1_ref[...] = h1
    wpc = wpc_ref[...]
    for t in range(T):
        rt2 = jnp.dot(h1[:, t * PC:(t + 1) * PC], wpc,
                      preferred_element_type=jnp.float32)
        rs_ref[:, t:t + 1] = rt2[:, 0:1]
        rs_ref[:, t + T:t + T + 1] = rt2[:, 1:2]


def _h1r(z0, z1, XW, dinv1, b1, Wp_rel, Wp_root):
    b1t = jnp.tile(b1, T)[None]
    wpc = jnp.concatenate([Wp_rel, Wp_root], axis=1)
    rb = pl.BlockSpec((_RB2, T * PC), lambda i: (i, 0))
    return pl.pallas_call(
        _h1r_body,
        grid=(N // _RB2,),
        in_specs=[rb, rb, rb,
                  pl.BlockSpec((_RB2, 1), lambda i: (i, 0)),
                  pl.BlockSpec((1, T * PC), lambda i: (0, 0)),
                  pl.BlockSpec((PC, 2), lambda i: (0, 0))],
        out_specs=[rb, pl.BlockSpec((_RB2, 2 * T), lambda i: (i, 0))],
        out_shape=[jax.ShapeDtypeStruct((N, T * PC), jnp.float32),
                   jax.ShapeDtypeStruct((N, 2 * T), jnp.float32)],
    )(z0, z1, XW, dinv1[:, None], b1t, wpc)


def _prep2_body(h1_ref, sc_ref, mk_ref, dg0_ref, dg1_ref,
                xp_ref, g_ref, dv2_ref):
    mk = mk_ref[...]
    sc = sc_ref[...]
    dg = dg0_ref[...] + dg1_ref[...]
    dv2 = mk * lax.rsqrt(jnp.maximum(mk * dg + mk, 1.0))
    dv2_ref[...] = dv2
    for t in range(T):
        xpt = h1_ref[:, t * PC:(t + 1) * PC] * (sc[:, t:t + 1] * mk[:, t:t + 1])
        xp_ref[:, t * PC:(t + 1) * PC] = xpt
        g_ref[:, t * PC:(t + 1) * PC] = dv2[:, t:t + 1] * xpt


def _prep2(h1, score, mask, dg0, dg1):
    rb = pl.BlockSpec((_RB2, T * PC), lambda i: (i, 0))
    tb = pl.BlockSpec((_RB2, T), lambda i: (i, 0))
    return pl.pallas_call(
        _prep2_body,
        grid=(N // _RB2,),
        in_specs=[rb, tb, tb, tb, tb],
        out_specs=[rb, rb, tb],
        out_shape=[jax.ShapeDtypeStruct((N, T * PC), jnp.float32),
                   jax.ShapeDtypeStruct((N, T * PC), jnp.float32),
                   jax.ShapeDtypeStruct((N, T), jnp.float32)],
    )(h1, score, mask, dg0, dg1)


def _pool_body(z0_ref, z1_ref, xp_ref, dv2_ref, mk_ref, w2_ref, b2_ref,
               bk_ref, sums_ref, cnt_ref):
    i = pl.program_id(0)
    w2 = w2_ref[...]
    b2 = b2_ref[...]
    mk = mk_ref[...]
    dv2 = dv2_ref[...]
    z = z0_ref[...] + z1_ref[...]
    onehot = (bk_ref[...] == lax.broadcasted_iota(jnp.int32, (_RB2, B), 1)
              ).astype(jnp.float32)
    cols = []
    for t in range(T):
        dvt = dv2[:, t:t + 1]
        mkt = mk[:, t:t + 1]
        pre = (dvt * z[:, t * PC:(t + 1) * PC]
               + (dvt * dvt * mkt) * xp_ref[:, t * PC:(t + 1) * PC])
        h2t = jax.nn.relu(
            (jnp.dot(pre, w2, preferred_element_type=jnp.float32) + b2) * mkt)
        cols.append(h2t)
    h2 = jnp.concatenate(cols, axis=1)
    dn = (((0,), (0,)), ((), ()))
    ps = lax.dot_general(onehot, h2, dn,
                         preferred_element_type=jnp.float32)
    pc = lax.dot_general(onehot, mk, dn,
                         preferred_element_type=jnp.float32)

    @pl.when(i == 0)
    def _():
        sums_ref[...] = ps
        cnt_ref[...] = pc

    @pl.when(i > 0)
    def _():
        sums_ref[...] += ps
        cnt_ref[...] += pc


def _pool(z0, z1, xp, dv2, mask, W2, b2, batch):
    rb = pl.BlockSpec((_RB2, T * PC), lambda i: (i, 0))
    tb = pl.BlockSpec((_RB2, T), lambda i: (i, 0))
    return pl.pallas_call(
        _pool_body,
        grid=(N // _RB2,),
        in_specs=[rb, rb, rb, tb, tb,
                  pl.BlockSpec((PC, H), lambda i: (0, 0)),
                  pl.BlockSpec((1, H), lambda i: (0, 0)),
                  pl.BlockSpec((_RB2, 1), lambda i: (i, 0))],
        out_specs=[pl.BlockSpec((B, T * H), lambda i: (0, 0)),
                   pl.BlockSpec((B, T), lambda i: (0, 0))],
        out_shape=[jax.ShapeDtypeStruct((B, T * H), jnp.float32),
                   jax.ShapeDtypeStruct((B, T), jnp.float32)],
    )(z0, z1, xp, dv2, mask, W2, b2[None], batch[:, None])


_RB = 128
_CB = 512


def _rank_body(score_ref, scoreT_ref, batch_ref, batchT_ref, gs_ref, ge_ref,
               kp_ref, out_ref):
    pid = pl.program_id(0)
    r0 = pid * _RB
    rows = score_ref[pl.ds(r0, _RB), :]
    b_i = batch_ref[pl.ds(r0, _RB), :]
    kp = kp_ref[pl.ds(r0, _RB), :]
    c_lo = jnp.min(gs_ref[pl.ds(r0, _RB), :])
    c_hi = jnp.max(ge_ref[pl.ds(r0, _RB), :])
    c_start = (c_lo // _CB) * _CB
    n_chunks = (c_hi - c_start + _CB - 1) // _CB

    iota_r = lax.broadcasted_iota(jnp.int32, (_RB, _CB), 0) + r0
    iota_c0 = lax.broadcasted_iota(jnp.int32, (_RB, _CB), 1)

    def chunk(k, acc):
        c = c_start + k * _CB
        b_j = batchT_ref[:, pl.ds(c, _CB)]
        same = b_i == b_j
        jlt = (iota_c0 + c) < iota_r
        cols = []
        for t in range(T):
            s_i = rows[:, t:t + 1]
            s_j = scoreT_ref[t:t + 1, pl.ds(c, _CB)]
            hit = jnp.logical_and(
                same,
                jnp.logical_or(s_j > s_i,
                               jnp.logical_and(s_j == s_i, jlt)))
            cols.append(jnp.sum(hit.astype(jnp.int32), axis=1, keepdims=True))
        return acc + jnp.concatenate(cols, axis=1)

    acc = lax.fori_loop(0, n_chunks, chunk, jnp.zeros((_RB, T), jnp.int32))
    out_ref[...] = (acc < kp).astype(jnp.float32)


def _topk_mask(score, batch, starts, counts, k_per):
    gs = starts[batch]
    ge = gs + counts[batch]
    kp = k_per[batch]
    scp = jnp.pad(score, ((0, _NPAD - N), (0, 0)))
    bp_ = jnp.pad(batch, (0, _NPAD - N), constant_values=127)
    gsp = jnp.pad(gs, (0, _NPAD - N), constant_values=N)
    gep = jnp.pad(ge, (0, _NPAD - N), constant_values=N)
    kpp = jnp.pad(kp, (0, _NPAD - N))
    full = lambda s: pl.BlockSpec(s, lambda i: (0, 0))
    mask = pl.pallas_call(
        _rank_body,
        grid=(_NPAD // _RB,),
        in_specs=[full((_NPAD, T)), full((T, _NPAD)), full((_NPAD, 1)),
                  full((1, _NPAD)), full((_NPAD, 1)), full((_NPAD, 1)),
                  full((_NPAD, 1))],
        out_specs=pl.BlockSpec((_RB, T), lambda i: (i, 0)),
        out_shape=jax.ShapeDtypeStruct((_NPAD, T), jnp.float32),
    )(scp, scp.T, bp_[:, None], bp_[None, :], gsp[:, None], gep[:, None],
      kpp[:, None])
    return mask[:N]


def _lstm_body(sums_ref, cnt_ref, wih_ref, whh_ref, bias_ref, wo_ref, bo_ref,
               out_ref):
    hs = jnp.zeros((B, H), jnp.float32)
    cs = jnp.zeros((B, H), jnp.float32)
    wih = wih_ref[...]
    whh = whh_ref[...]
    bias = bias_ref[...]
    cnt = jnp.maximum(cnt_ref[...], 1.0)
    for t in range(T):
        seq_t = sums_ref[:, t * H:(t + 1) * H] / cnt[:, t:t + 1]
        g = (jnp.dot(seq_t, wih, preferred_element_type=jnp.float32)
             + jnp.dot(hs, whh, preferred_element_type=jnp.float32) + bias)
        i = jax.nn.sigmoid(g[:, 0:H])
        f = jax.nn.sigmoid(g[:, H:2 * H])
        gg = jnp.tanh(g[:, 2 * H:3 * H])
        o = jax.nn.sigmoid(g[:, 3 * H:4 * H])
        cs = f * cs + i * gg
        hs = o * jnp.tanh(cs)
    out_ref[...] = jax.nn.sigmoid(
        jnp.dot(hs, wo_ref[...], preferred_element_type=jnp.float32) + bo_ref[...])


def _lstm_head(sums, cnt, W_ih, W_hh, b_ih, b_hh, Wo, bo):
    bias = (b_ih + b_hh).reshape(1, 4 * H)
    return pl.pallas_call(
        _lstm_body,
        out_shape=jax.ShapeDtypeStruct((B, 1), jnp.float32),
    )(sums, cnt, W_ih.T, W_hh.T, bias, Wo, bo.reshape(1, 1))


_HI = jax.lax.Precision.HIGHEST


def kernel(x, edge_index, edge_weight, batch, W1, b1, Wp_rel, bp, Wp_root,
           W2, b2, W_ih, W_hh, b_ih, b_hh, Wo, bo):
    src = edge_index[0]
    dst = edge_index[1]
    ew = edge_weight
    counts = jnp.bincount(batch, length=B)
    starts = jnp.concatenate(
        [jnp.zeros((1,), counts.dtype), jnp.cumsum(counts)])[:-1]
    k_per = jnp.ceil(RATIO * counts.astype(jnp.float32)).astype(jnp.int32)

    degA = _edge_pass(None, src, dst, ew, 16, True, False)
    deg1 = degA[0, :N, 0] + degA[1, :N, 0] + 1.0
    dinv1 = lax.rsqrt(deg1)

    XW, M1 = _prep1(x, W1, dinv1)
    Z1 = _edge_pass(M1, src, dst, ew, 128, True, True)
    h1, rs = _h1r(Z1[0, :N], Z1[1, :N], XW, dinv1, b1, Wp_rel, Wp_root)

    agg = _edge_pass(rs, src, dst, ew, 16, False, True)
    score = jnp.tanh(agg[0, :N, :T] + agg[1, :N, :T] + bp[0] + rs[:, T:])

    mask = _topk_mask(score, batch, starts, counts, k_per)

    maskt = jnp.pad(mask, ((0, 0), (0, 16 - T)))
    DEG2 = _edge_pass(maskt, src, dst, ew, 16, True, True)
    xp, G, dinv2 = _prep2(h1, score, mask, DEG2[0, :N, :T], DEG2[1, :N, :T])

    Z2 = _edge_pass(G, src, dst, ew, 128, True, True)
    sums, cnt = _pool(Z2[0, :N], Z2[1, :N], xp, dinv2, mask, W2, b2, batch)

    return _lstm_head(sums, cnt, W_ih, W_hh, b_ih, b_hh, Wo, bo)


# speedup vs baseline: 81.2126x; 1.4911x over previous
import functools

import jax
import jax.numpy as jnp
from jax import lax
from jax.experimental import pallas as pl
from jax.experimental.pallas import tpu as pltpu
from jax.experimental.pallas import tpu_sc as plsc

N = 10000
E = 320000
T = 8
B = 64
F_IN = 128
PC = 16
H = 64
RATIO = 0.8

_NC = 2
_NS = 16
_NPAD = 10240
_EPW = E // (_NC * _NS)
_K = 80
_ITERS = _EPW // _K
_RPT = _NPAD // _NS


@functools.lru_cache(maxsize=None)
def _edge_pass_fn(D, scale, gather):
    nvr = D // 16
    mesh = plsc.VectorSubcoreMesh(core_axis_name="c", subcore_axis_name="s")

    nbuf = 2 if gather else 1
    scratch = []
    for _ in range(nbuf):
        scratch.append(pltpu.VMEM((_K,), jnp.int32))
        if gather:
            scratch.append(pltpu.VMEM((_K,), jnp.int32))
        if scale:
            scratch.append(pltpu.VMEM((_K,), jnp.float32))
        scratch.append(pltpu.VMEM((_K, D), jnp.float32))
        scratch.append(pltpu.SemaphoreType.DMA)
        scratch.append(pltpu.SemaphoreType.DMA)
    scratch.append(pltpu.VMEM_SHARED((_NPAD, D), jnp.float32))

    def body(*refs):
        refs = list(refs)
        table = refs.pop(0) if gather else None
        srcr = refs.pop(0) if gather else None
        dstr = refs.pop(0)
        ewr = refs.pop(0) if scale else None
        out = refs.pop(0)
        bufs = []
        for _ in range(nbuf):
            bufs.append(dict(
                idx_d=refs.pop(0),
                idx_s=refs.pop(0) if gather else None,
                ew_v=refs.pop(0) if scale else None,
                rows=refs.pop(0),
                sem_l=refs.pop(0),
                sem_g=refs.pop(0),
            ))
        acc = refs.pop(0)
        idx_d = bufs[0]["idx_d"]
        ew_v = bufs[0]["ew_v"]
        rows = bufs[0]["rows"]

        c = lax.axis_index("c")
        s = lax.axis_index("s")
        ebase = (c * _NS + s) * _EPW
        tbase = s * _RPT

        def zrow(e, carry):
            for j in range(nvr):
                rows[e, pl.ds(j * 16, 16)] = jnp.zeros((16,), jnp.float32)
            return carry
        lax.fori_loop(0, _K, zrow, 0)
        for kc in range(_RPT // _K):
            pltpu.sync_copy(rows, acc.at[pl.ds(tbase + kc * _K, _K)])
        plsc.subcore_barrier()

        lane0 = lax.iota(jnp.int32, 16) == 0

        def issue_lin(k, b):
            base = ebase + k * _K
            bf = bufs[b]
            pltpu.async_copy(dstr.at[pl.ds(base, _K)], bf["idx_d"], bf["sem_l"])
            if gather:
                pltpu.async_copy(srcr.at[pl.ds(base, _K)], bf["idx_s"],
                                 bf["sem_l"])
            if scale:
                pltpu.async_copy(ewr.at[pl.ds(base, _K)], bf["ew_v"],
                                 bf["sem_l"])

        def wait_lin(b):
            bf = bufs[b]
            pltpu.make_async_copy(dstr.at[pl.ds(0, _K)], bf["idx_d"],
                                  bf["sem_l"]).wait()
            if gather:
                pltpu.make_async_copy(srcr.at[pl.ds(0, _K)], bf["idx_s"],
                                      bf["sem_l"]).wait()
            if scale:
                pltpu.make_async_copy(ewr.at[pl.ds(0, _K)], bf["ew_v"],
                                      bf["sem_l"]).wait()

        def issue_gather(b):
            bf = bufs[b]
            pltpu.async_copy(table.at[bf["idx_s"]], bf["rows"], bf["sem_g"])

        def wait_gather(b):
            bf = bufs[b]
            pltpu.make_async_copy(table.at[bf["idx_s"]], bf["rows"],
                                  bf["sem_g"]).wait()

        def process(b):
            bf = bufs[b]
            r = bf["rows"]
            if scale:
                w_ref = bf["ew_v"]

                def sc_body(g, cr):
                    wvec = w_ref[pl.ds(g * 16, 16)]
                    for l in range(16):
                        e = g * 16 + l
                        w = wvec[l]
                        for j in range(nvr):
                            sl = pl.ds(j * 16, 16)
                            r[e, sl] = r[e, sl] * w
                    return cr
                lax.fori_loop(0, _K // 16, sc_body, 0)
            pltpu.sync_copy(r, acc.at[bf["idx_d"]], add=True)

        if gather:
            issue_lin(0, 0)
            issue_lin(1, 1)
            wait_lin(0)
            issue_gather(0)

            def pair(g, cr):
                k2 = 2 * g + 2
                wait_lin(1)
                issue_gather(1)
                wait_gather(0)
                process(0)

                @pl.when(k2 < _ITERS)
                def _():
                    issue_lin(k2, 0)
                wait_gather(1)
                process(1)

                @pl.when(k2 + 1 < _ITERS)
                def _():
                    issue_lin(k2 + 1, 1)

                @pl.when(k2 < _ITERS)
                def _():
                    wait_lin(0)
                    issue_gather(0)
                return cr
            lax.fori_loop(0, _ITERS // 2, pair, 0)
            if _ITERS % 2 == 1:
                wait_gather(0)
                process(0)
        else:
            def step(it, carry):
                base = ebase + it * _K
                pltpu.sync_copy(dstr.at[pl.ds(base, _K)], idx_d)
                pltpu.sync_copy(ewr.at[pl.ds(base, _K)], ew_v)

                def a_body(g, cr):
                    wvec = ew_v[pl.ds(g * 16, 16)]
                    for l in range(16):
                        e = g * 16 + l
                        rows[e, pl.ds(0, 16)] = jnp.where(lane0, wvec[l], 0.0)
                    return cr
                lax.fori_loop(0, _K // 16, a_body, 0)
                pltpu.sync_copy(rows, acc.at[idx_d], add=True)
                return carry
            lax.fori_loop(0, _ITERS, step, 0)
        plsc.subcore_barrier()
        pltpu.sync_copy(acc.at[pl.ds(tbase, _RPT)],
                        out.at[c, pl.ds(tbase, _RPT)])

    return pl.kernel(
        body, mesh=mesh,
        out_type=jax.ShapeDtypeStruct((_NC, _NPAD, D), jnp.float32),
        scratch_types=scratch,
        compiler_params=pltpu.CompilerParams(use_tc_tiling_on_sc=False),
    )


def _edge_pass(table, src, dst, ew, D, scale, gather):
    fn = _edge_pass_fn(D, scale, gather)
    args = []
    if gather:
        args += [table, src]
    args += [dst]
    if scale:
        args += [ew]
    return fn(*args)


_RB2 = 400


def _row_specs(shapes):
    specs = []
    for s in shapes:
        if s[0] is None:
            full = tuple(1 if d is None else d for d in s)
            specs.append(pl.BlockSpec(
                tuple(d for d in full),
                lambda i: tuple(0 for _ in full)))
        else:
            specs.append(pl.BlockSpec(s, lambda i: (i,) + (0,) * (len(s) - 1)))
    return specs


def _prep1_body(x_ref, w1_ref, dinv_ref, xw_ref, m1_ref):
    w1 = w1_ref[...]
    dv = dinv_ref[...]
    for t in range(T):
        xw = jnp.dot(x_ref[t], w1, preferred_element_type=jnp.float32)
        xw_ref[:, t * PC:(t + 1) * PC] = xw
        m1_ref[:, t * PC:(t + 1) * PC] = dv * xw


def _prep1(x, W1, dinv1):
    return pl.pallas_call(
        _prep1_body,
        grid=(N // _RB2,),
        in_specs=[
            pl.BlockSpec((T, _RB2, F_IN), lambda i: (0, i, 0)),
            pl.BlockSpec((F_IN, PC), lambda i: (0, 0)),
            pl.BlockSpec((_RB2, 1), lambda i: (i, 0)),
        ],
        out_specs=[pl.BlockSpec((_RB2, T * PC), lambda i: (i, 0))] * 2,
        out_shape=[jax.ShapeDtypeStruct((N, T * PC), jnp.float32)] * 2,
    )(x, W1, dinv1[:, None])


def _h1r_body(z0_ref, z1_ref, xw_ref, dinv_ref, b1t_ref, wpc_ref,
              h1_ref, rs_ref):
    dv = dinv_ref[...]
    h1 = jax.nn.relu(dv * (z0_ref[...] + z1_ref[...])
                     + (dv * dv) * xw_ref[...] + b1t_ref[...])
    h1_ref[...] = h1
    wpc = wpc_ref[...]
    for t in range(T):
        rt2 = jnp.dot(h1[:, t * PC:(t + 1) * PC], wpc,
                      preferred_element_type=jnp.float32)
        rs_ref[:, t:t + 1] = rt2[:, 0:1]
        rs_ref[:, t + T:t + T + 1] = rt2[:, 1:2]


def _h1r(z0, z1, XW, dinv1, b1, Wp_rel, Wp_root):
    b1t = jnp.tile(b1, T)[None]
    wpc = jnp.concatenate([Wp_rel, Wp_root], axis=1)
    rb = pl.BlockSpec((_RB2, T * PC), lambda i: (i, 0))
    return pl.pallas_call(
        _h1r_body,
        grid=(N // _RB2,),
        in_specs=[rb, rb, rb,
                  pl.BlockSpec((_RB2, 1), lambda i: (i, 0)),
                  pl.BlockSpec((1, T * PC), lambda i: (0, 0)),
                  pl.BlockSpec((PC, 2), lambda i: (0, 0))],
        out_specs=[rb, pl.BlockSpec((_RB2, 2 * T), lambda i: (i, 0))],
        out_shape=[jax.ShapeDtypeStruct((N, T * PC), jnp.float32),
                   jax.ShapeDtypeStruct((N, 2 * T), jnp.float32)],
    )(z0, z1, XW, dinv1[:, None], b1t, wpc)


def _prep2_body(h1_ref, sc_ref, mk_ref, dg0_ref, dg1_ref,
                xp_ref, g_ref, dv2_ref):
    mk = mk_ref[...]
    sc = sc_ref[...]
    dg = dg0_ref[...] + dg1_ref[...]
    dv2 = mk * lax.rsqrt(jnp.maximum(mk * dg + mk, 1.0))
    dv2_ref[...] = dv2
    for t in range(T):
        xpt = h1_ref[:, t * PC:(t + 1) * PC] * (sc[:, t:t + 1] * mk[:, t:t + 1])
        xp_ref[:, t * PC:(t + 1) * PC] = xpt
        g_ref[:, t * PC:(t + 1) * PC] = dv2[:, t:t + 1] * xpt


def _prep2(h1, score, mask, dg0, dg1):
    rb = pl.BlockSpec((_RB2, T * PC), lambda i: (i, 0))
    tb = pl.BlockSpec((_RB2, T), lambda i: (i, 0))
    return pl.pallas_call(
        _prep2_body,
        grid=(N // _RB2,),
        in_specs=[rb, tb, tb, tb, tb],
        out_specs=[rb, rb, tb],
        out_shape=[jax.ShapeDtypeStruct((N, T * PC), jnp.float32),
                   jax.ShapeDtypeStruct((N, T * PC), jnp.float32),
                   jax.ShapeDtypeStruct((N, T), jnp.float32)],
    )(h1, score, mask, dg0, dg1)


def _pool_body(z0_ref, z1_ref, xp_ref, dv2_ref, mk_ref, w2_ref, b2_ref,
               bk_ref, sums_ref, cnt_ref):
    i = pl.program_id(0)
    w2 = w2_ref[...]
    b2 = b2_ref[...]
    mk = mk_ref[...]
    dv2 = dv2_ref[...]
    z = z0_ref[...] + z1_ref[...]
    onehot = (bk_ref[...] == lax.broadcasted_iota(jnp.int32, (_RB2, B), 1)
              ).astype(jnp.float32)
    cols = []
    for t in range(T):
        dvt = dv2[:, t:t + 1]
        mkt = mk[:, t:t + 1]
        pre = (dvt * z[:, t * PC:(t + 1) * PC]
               + (dvt * dvt * mkt) * xp_ref[:, t * PC:(t + 1) * PC])
        h2t = jax.nn.relu(
            (jnp.dot(pre, w2, preferred_element_type=jnp.float32) + b2) * mkt)
        cols.append(h2t)
    h2 = jnp.concatenate(cols, axis=1)
    dn = (((0,), (0,)), ((), ()))
    ps = lax.dot_general(onehot, h2, dn,
                         preferred_element_type=jnp.float32)
    pc = lax.dot_general(onehot, mk, dn,
                         preferred_element_type=jnp.float32)

    @pl.when(i == 0)
    def _():
        sums_ref[...] = ps
        cnt_ref[...] = pc

    @pl.when(i > 0)
    def _():
        sums_ref[...] += ps
        cnt_ref[...] += pc


def _pool(z0, z1, xp, dv2, mask, W2, b2, batch):
    rb = pl.BlockSpec((_RB2, T * PC), lambda i: (i, 0))
    tb = pl.BlockSpec((_RB2, T), lambda i: (i, 0))
    return pl.pallas_call(
        _pool_body,
        grid=(N // _RB2,),
        in_specs=[rb, rb, rb, tb, tb,
                  pl.BlockSpec((PC, H), lambda i: (0, 0)),
                  pl.BlockSpec((1, H), lambda i: (0, 0)),
                  pl.BlockSpec((_RB2, 1), lambda i: (i, 0))],
        out_specs=[pl.BlockSpec((B, T * H), lambda i: (0, 0)),
                   pl.BlockSpec((B, T), lambda i: (0, 0))],
        out_shape=[jax.ShapeDtypeStruct((B, T * H), jnp.float32),
                   jax.ShapeDtypeStruct((B, T), jnp.float32)],
    )(z0, z1, xp, dv2, mask, W2, b2[None], batch[:, None])


_RB = 128
_CB = 512


def _rank_body(score_ref, scoreT_ref, batch_ref, batchT_ref, gs_ref, ge_ref,
               kp_ref, out_ref):
    pid = pl.program_id(0)
    r0 = pid * _RB
    rows = score_ref[pl.ds(r0, _RB), :]
    b_i = batch_ref[pl.ds(r0, _RB), :]
    kp = kp_ref[pl.ds(r0, _RB), :]
    c_lo = jnp.min(gs_ref[pl.ds(r0, _RB), :])
    c_hi = jnp.max(ge_ref[pl.ds(r0, _RB), :])
    c_start = (c_lo // _CB) * _CB
    n_chunks = (c_hi - c_start + _CB - 1) // _CB

    iota_r = lax.broadcasted_iota(jnp.int32, (_RB, _CB), 0) + r0
    iota_c0 = lax.broadcasted_iota(jnp.int32, (_RB, _CB), 1)

    def chunk(k, acc):
        c = c_start + k * _CB
        b_j = batchT_ref[:, pl.ds(c, _CB)]
        same = b_i == b_j
        jlt = (iota_c0 + c) < iota_r
        cols = []
        for t in range(T):
            s_i = rows[:, t:t + 1]
            s_j = scoreT_ref[t:t + 1, pl.ds(c, _CB)]
            hit = jnp.logical_and(
                same,
                jnp.logical_or(s_j > s_i,
                               jnp.logical_and(s_j == s_i, jlt)))
            cols.append(jnp.sum(hit.astype(jnp.int32), axis=1, keepdims=True))
        return acc + jnp.concatenate(cols, axis=1)

    acc = lax.fori_loop(0, n_chunks, chunk, jnp.zeros((_RB, T), jnp.int32))
    out_ref[...] = (acc < kp).astype(jnp.float32)


def _topk_mask(score, batch, starts, counts, k_per):
    gs = starts[batch]
    ge = gs + counts[batch]
    kp = k_per[batch]
    scp = jnp.pad(score, ((0, _NPAD - N), (0, 0)))
    bp_ = jnp.pad(batch, (0, _NPAD - N), constant_values=127)
    gsp = jnp.pad(gs, (0, _NPAD - N), constant_values=N)
    gep = jnp.pad(ge, (0, _NPAD - N), constant_values=N)
    kpp = jnp.pad(kp, (0, _NPAD - N))
    full = lambda s: pl.BlockSpec(s, lambda i: (0, 0))
    mask = pl.pallas_call(
        _rank_body,
        grid=(_NPAD // _RB,),
        in_specs=[full((_NPAD, T)), full((T, _NPAD)), full((_NPAD, 1)),
                  full((1, _NPAD)), full((_NPAD, 1)), full((_NPAD, 1)),
                  full((_NPAD, 1))],
        out_specs=pl.BlockSpec((_RB, T), lambda i: (i, 0)),
        out_shape=jax.ShapeDtypeStruct((_NPAD, T), jnp.float32),
    )(scp, scp.T, bp_[:, None], bp_[None, :], gsp[:, None], gep[:, None],
      kpp[:, None])
    return mask[:N]


def _lstm_body(sums_ref, cnt_ref, wih_ref, whh_ref, bias_ref, wo_ref, bo_ref,
               out_ref):
    hs = jnp.zeros((B, H), jnp.float32)
    cs = jnp.zeros((B, H), jnp.float32)
    wih = wih_ref[...]
    whh = whh_ref[...]
    bias = bias_ref[...]
    cnt = jnp.maximum(cnt_ref[...], 1.0)
    for t in range(T):
        seq_t = sums_ref[:, t * H:(t + 1) * H] / cnt[:, t:t + 1]
        g = (jnp.dot(seq_t, wih, preferred_element_type=jnp.float32)
             + jnp.dot(hs, whh, preferred_element_type=jnp.float32) + bias)
        i = jax.nn.sigmoid(g[:, 0:H])
        f = jax.nn.sigmoid(g[:, H:2 * H])
        gg = jnp.tanh(g[:, 2 * H:3 * H])
        o = jax.nn.sigmoid(g[:, 3 * H:4 * H])
        cs = f * cs + i * gg
        hs = o * jnp.tanh(cs)
    out_ref[...] = jax.nn.sigmoid(
        jnp.dot(hs, wo_ref[...], preferred_element_type=jnp.float32) + bo_ref[...])


def _lstm_head(sums, cnt, W_ih, W_hh, b_ih, b_hh, Wo, bo):
    bias = (b_ih + b_hh).reshape(1, 4 * H)
    return pl.pallas_call(
        _lstm_body,
        out_shape=jax.ShapeDtypeStruct((B, 1), jnp.float32),
    )(sums, cnt, W_ih.T, W_hh.T, bias, Wo, bo.reshape(1, 1))


_HI = jax.lax.Precision.HIGHEST


def kernel(x, edge_index, edge_weight, batch, W1, b1, Wp_rel, bp, Wp_root,
           W2, b2, W_ih, W_hh, b_ih, b_hh, Wo, bo):
    src = edge_index[0]
    dst = edge_index[1]
    ew = edge_weight
    counts = jnp.bincount(batch, length=B)
    starts = jnp.concatenate(
        [jnp.zeros((1,), counts.dtype), jnp.cumsum(counts)])[:-1]
    k_per = jnp.ceil(RATIO * counts.astype(jnp.float32)).astype(jnp.int32)

    degA = _edge_pass(None, src, dst, ew, 16, True, False)
    deg1 = degA[0, :N, 0] + degA[1, :N, 0] + 1.0
    dinv1 = lax.rsqrt(deg1)

    XW, M1 = _prep1(x, W1, dinv1)
    Z1 = _edge_pass(M1, src, dst, ew, 128, True, True)
    h1, rs = _h1r(Z1[0, :N], Z1[1, :N], XW, dinv1, b1, Wp_rel, Wp_root)

    agg = _edge_pass(rs, src, dst, ew, 16, False, True)
    score = jnp.tanh(agg[0, :N, :T] + agg[1, :N, :T] + bp[0] + rs[:, T:])

    mask = _topk_mask(score, batch, starts, counts, k_per)

    maskt = jnp.pad(mask, ((0, 0), (0, 16 - T)))
    DEG2 = _edge_pass(maskt, src, dst, ew, 16, True, True)
    xp, G, dinv2 = _prep2(h1, score, mask, DEG2[0, :N, :T], DEG2[1, :N, :T])

    Z2 = _edge_pass(G, src, dst, ew, 128, True, True)
    sums, cnt = _pool(Z2[0, :N], Z2[1, :N], xp, dinv2, mask, W2, b2, batch)

    return _lstm_head(sums, cnt, W_ih, W_hh, b_ih, b_hh, Wo, bo)
